# baseline (device time: 31662 ns/iter reference)
import jax
import jax.numpy as jnp
from jax import lax
from jax.experimental import pallas as pl
from jax.experimental.pallas import tpu as pltpu

N_DEV = 4
B, SQ, SKV, HQ_LOCAL, DH = 2, 128, 128, 4, 64
D_MODEL = 512


def kernel(x, Wq, K_ext, V_ext, Wo):
    h0 = HQ_LOCAL * lax.axis_index("i")
    K_loc = lax.dynamic_slice_in_dim(K_ext, h0, HQ_LOCAL, axis=2)
    V_loc = lax.dynamic_slice_in_dim(V_ext, h0, HQ_LOCAL, axis=2)

    def body(x_ref, wq_ref, k_ref, v_ref, wo_ref, out_ref,
             comm_ref, send_sems, recv_sems):
        my_pos = lax.axis_index("i")
        left = lax.rem(my_pos + N_DEV - 1, N_DEV)
        right = lax.rem(my_pos + 1, N_DEV)

        barrier_sem = pltpu.get_barrier_semaphore()
        for nbr in (left, right):
            pl.semaphore_signal(
                barrier_sem, inc=1,
                device_id=(nbr,), device_id_type=pl.DeviceIdType.MESH,
            )
        pl.semaphore_wait(barrier_sem, 2)

        wq = wq_ref[:, :].astype(jnp.bfloat16)
        wo = wo_ref[:, :].astype(jnp.bfloat16)
        for b in range(B):
            xb = x_ref[b, :, :].astype(jnp.bfloat16)
            qb = jnp.dot(xb, wq, preferred_element_type=jnp.float32)
            ctxs = []
            for h in range(HQ_LOCAL):
                qh = qb[:, h * DH:(h + 1) * DH].astype(jnp.bfloat16)
                kh = k_ref[b, :, h, :].astype(jnp.bfloat16)
                vh = v_ref[b, :, h, :].astype(jnp.bfloat16)
                s = jnp.dot(qh, kh.T, preferred_element_type=jnp.float32) * 0.125
                s = s - jnp.max(s, axis=-1, keepdims=True)
                w = jnp.exp(s)
                w = w / jnp.sum(w, axis=-1, keepdims=True)
                ctxs.append(jnp.dot(w.astype(jnp.bfloat16), vh,
                                    preferred_element_type=jnp.float32))
            ctx_b = jnp.concatenate(ctxs, axis=-1)
            pb = jnp.dot(ctx_b.astype(jnp.bfloat16), wo,
                         preferred_element_type=jnp.float32)
            out_ref[b, :, :] = pb
            comm_ref[0, b, :, :] = pb

        for hop in range(N_DEV - 1):
            send_slot = hop % 2
            recv_slot = (hop + 1) % 2
            rdma = pltpu.make_async_remote_copy(
                src_ref=comm_ref.at[send_slot],
                dst_ref=comm_ref.at[recv_slot],
                send_sem=send_sems.at[send_slot],
                recv_sem=recv_sems.at[recv_slot],
                device_id=(right,),
                device_id_type=pl.DeviceIdType.MESH,
            )
            rdma.start()
            rdma.wait()
            out_ref[:, :, :] = out_ref[:, :, :] + comm_ref[recv_slot]

    return pl.pallas_call(
        body,
        out_shape=jax.ShapeDtypeStruct((B, SQ, D_MODEL), jnp.float32),
        in_specs=[pl.BlockSpec(memory_space=pltpu.VMEM)] * 5,
        out_specs=pl.BlockSpec(memory_space=pltpu.VMEM),
        scratch_shapes=[
            pltpu.VMEM((2, B, SQ, D_MODEL), jnp.float32),
            pltpu.SemaphoreType.DMA((2,)),
            pltpu.SemaphoreType.DMA((2,)),
        ],
        compiler_params=pltpu.CompilerParams(collective_id=0),
    )(x, Wq, K_loc, V_loc, Wo)


# device time: 18269 ns/iter; 1.7331x vs baseline; 1.7331x over previous
import jax
import jax.numpy as jnp
from jax import lax
from jax.experimental import pallas as pl
from jax.experimental.pallas import tpu as pltpu

N_DEV = 4
B, SQ, SKV, HQ_LOCAL, DH = 2, 128, 128, 4, 64
D_MODEL = 512


def kernel(x, Wq, K_ext, V_ext, Wo):
    h0 = HQ_LOCAL * lax.axis_index("i")
    K_loc = lax.dynamic_slice_in_dim(K_ext, h0, HQ_LOCAL, axis=2)
    V_loc = lax.dynamic_slice_in_dim(V_ext, h0, HQ_LOCAL, axis=2)

    def body(x_ref, wq_ref, k_ref, v_ref, wo_ref, out_ref,
             send_ref, recv_ref, send_sems, recv_sems):
        my_pos = lax.axis_index("i")
        partner_a = my_pos ^ 1
        partner_b = 3 - my_pos

        barrier_sem = pltpu.get_barrier_semaphore()
        for nbr in (partner_a, partner_b):
            pl.semaphore_signal(
                barrier_sem, inc=1,
                device_id=(nbr,), device_id_type=pl.DeviceIdType.MESH,
            )
        pl.semaphore_wait(barrier_sem, 2)

        wq = wq_ref[:, :].astype(jnp.bfloat16)
        wo = wo_ref[:, :].astype(jnp.bfloat16)
        for b in range(B):
            xb = x_ref[b, :, :].astype(jnp.bfloat16)
            qb = jnp.dot(xb, wq, preferred_element_type=jnp.float32)
            ctxs = []
            for h in range(HQ_LOCAL):
                qh = qb[:, h * DH:(h + 1) * DH].astype(jnp.bfloat16)
                kh = k_ref[b, :, h, :].astype(jnp.bfloat16)
                vh = v_ref[b, :, h, :].astype(jnp.bfloat16)
                s = jnp.dot(qh, kh.T, preferred_element_type=jnp.float32) * 0.125
                s = s - jnp.max(s, axis=-1, keepdims=True)
                w = jnp.exp(s)
                w = w / jnp.sum(w, axis=-1, keepdims=True)
                ctxs.append(jnp.dot(w.astype(jnp.bfloat16), vh,
                                    preferred_element_type=jnp.float32))
            ctx_b = jnp.concatenate(ctxs, axis=-1)
            pb = jnp.dot(ctx_b.astype(jnp.bfloat16), wo,
                         preferred_element_type=jnp.float32)
            out_ref[b, :, :] = pb
            send_ref[0, b, :, :] = pb.astype(jnp.bfloat16)

        rdma_a = pltpu.make_async_remote_copy(
            src_ref=send_ref.at[0],
            dst_ref=recv_ref.at[0],
            send_sem=send_sems.at[0],
            recv_sem=recv_sems.at[0],
            device_id=(partner_a,),
            device_id_type=pl.DeviceIdType.MESH,
        )
        rdma_a.start()
        rdma_a.wait()
        acc = out_ref[:, :, :] + recv_ref[0].astype(jnp.float32)
        out_ref[:, :, :] = acc
        send_ref[1] = acc.astype(jnp.bfloat16)

        rdma_b = pltpu.make_async_remote_copy(
            src_ref=send_ref.at[1],
            dst_ref=recv_ref.at[1],
            send_sem=send_sems.at[1],
            recv_sem=recv_sems.at[1],
            device_id=(partner_b,),
            device_id_type=pl.DeviceIdType.MESH,
        )
        rdma_b.start()
        rdma_b.wait()
        out_ref[:, :, :] = out_ref[:, :, :] + recv_ref[1].astype(jnp.float32)

    return pl.pallas_call(
        body,
        out_shape=jax.ShapeDtypeStruct((B, SQ, D_MODEL), jnp.float32),
        in_specs=[pl.BlockSpec(memory_space=pltpu.VMEM)] * 5,
        out_specs=pl.BlockSpec(memory_space=pltpu.VMEM),
        scratch_shapes=[
            pltpu.VMEM((2, B, SQ, D_MODEL), jnp.bfloat16),
            pltpu.VMEM((2, B, SQ, D_MODEL), jnp.bfloat16),
            pltpu.SemaphoreType.DMA((2,)),
            pltpu.SemaphoreType.DMA((2,)),
        ],
        compiler_params=pltpu.CompilerParams(collective_id=0),
    )(x, Wq, K_loc, V_loc, Wo)


# device time: 15952 ns/iter; 1.9848x vs baseline; 1.1452x over previous
import jax
import jax.numpy as jnp
from jax import lax
from jax.experimental import pallas as pl
from jax.experimental.pallas import tpu as pltpu

N_DEV = 4
B, SQ, SKV, HQ_LOCAL, DH = 2, 128, 128, 4, 64
D_MODEL = 512


def kernel(x, Wq, K_ext, V_ext, Wo):
    h0 = HQ_LOCAL * lax.axis_index("i")
    K_loc = lax.dynamic_slice_in_dim(K_ext, h0, HQ_LOCAL, axis=2).astype(jnp.bfloat16)
    V_loc = lax.dynamic_slice_in_dim(V_ext, h0, HQ_LOCAL, axis=2).astype(jnp.bfloat16)

    def body(x_ref, wq_ref, k_ref, v_ref, wo_ref, out_ref,
             acc_ref, send_ref, recv_ref, send_sems, recv_sems):
        my_pos = lax.axis_index("i")
        partner_a = my_pos ^ 1
        partner_b = 3 - my_pos

        barrier_sem = pltpu.get_barrier_semaphore()
        for nbr in (partner_a, partner_b):
            pl.semaphore_signal(
                barrier_sem, inc=1,
                device_id=(nbr,), device_id_type=pl.DeviceIdType.MESH,
            )

        def _exchange(stage, b, partner):
            return pltpu.make_async_remote_copy(
                src_ref=send_ref.at[stage, b],
                dst_ref=recv_ref.at[stage, b],
                send_sem=send_sems.at[stage, b],
                recv_sem=recv_sems.at[stage, b],
                device_id=(partner,),
                device_id_type=pl.DeviceIdType.MESH,
            )

        rdma_a = [_exchange(0, b, partner_a) for b in range(B)]
        rdma_b = [_exchange(1, b, partner_b) for b in range(B)]

        wq = wq_ref[:, :].astype(jnp.bfloat16)
        wo = wo_ref[:, :].astype(jnp.bfloat16)

        for b in range(B):
            xb = x_ref[b, :, :].astype(jnp.bfloat16)
            qb = jnp.dot(xb, wq, preferred_element_type=jnp.float32)
            ctxs = []
            for h in range(HQ_LOCAL):
                qh = qb[:, h * DH:(h + 1) * DH].astype(jnp.bfloat16)
                kh = k_ref[b, :, h, :]
                vh = v_ref[b, :, h, :]
                s = jnp.dot(qh, kh.T, preferred_element_type=jnp.float32) * 0.125
                s = s - jnp.max(s, axis=-1, keepdims=True)
                w = jnp.exp(s)
                w = w / jnp.sum(w, axis=-1, keepdims=True)
                ctxs.append(jnp.dot(w.astype(jnp.bfloat16), vh,
                                    preferred_element_type=jnp.float32))
            ctx_b = jnp.concatenate(ctxs, axis=-1)
            pb = jnp.dot(ctx_b.astype(jnp.bfloat16), wo,
                         preferred_element_type=jnp.float32)
            acc_ref[b, :, :] = pb
            send_ref[0, b, :, :] = pb.astype(jnp.bfloat16)
            if b == 0:
                pl.semaphore_wait(barrier_sem, 2)
            rdma_a[b].start()

        for b in range(B):
            rdma_a[b].wait()
            acc = acc_ref[b, :, :] + recv_ref[0, b].astype(jnp.float32)
            acc_ref[b, :, :] = acc
            send_ref[1, b, :, :] = acc.astype(jnp.bfloat16)
            rdma_b[b].start()

        for b in range(B):
            rdma_b[b].wait()
            out_ref[b, :, :] = (acc_ref[b, :, :]
                                + recv_ref[1, b].astype(jnp.float32)
                                ).astype(out_ref.dtype)

    return pl.pallas_call(
        body,
        out_shape=jax.ShapeDtypeStruct((B, SQ, D_MODEL), jnp.bfloat16),
        in_specs=[pl.BlockSpec(memory_space=pltpu.VMEM)] * 5,
        out_specs=pl.BlockSpec(memory_space=pltpu.VMEM),
        scratch_shapes=[
            pltpu.VMEM((B, SQ, D_MODEL), jnp.float32),
            pltpu.VMEM((2, B, SQ, D_MODEL), jnp.bfloat16),
            pltpu.VMEM((2, B, SQ, D_MODEL), jnp.bfloat16),
            pltpu.SemaphoreType.DMA((2, B)),
            pltpu.SemaphoreType.DMA((2, B)),
        ],
        compiler_params=pltpu.CompilerParams(collective_id=0),
    )(x, Wq, K_loc, V_loc, Wo)


# device time: 15207 ns/iter; 2.0821x vs baseline; 1.0490x over previous
import jax
import jax.numpy as jnp
from jax import lax
from jax.experimental import pallas as pl
from jax.experimental.pallas import tpu as pltpu

N_DEV = 4
B, SQ, SKV, HQ_LOCAL, DH = 2, 128, 128, 4, 64
D_MODEL = 512


def kernel(x, Wq, K_ext, V_ext, Wo):
    h0 = HQ_LOCAL * lax.axis_index("i")
    K_loc = lax.dynamic_slice_in_dim(K_ext, h0, HQ_LOCAL, axis=2)
    V_loc = lax.dynamic_slice_in_dim(V_ext, h0, HQ_LOCAL, axis=2)

    def body(x_ref, wq_ref, k_ref, v_ref, wo_ref, out_ref,
             acc_ref, send_ref, recv_ref, send_sems, recv_sems):
        my_pos = lax.axis_index("i")
        partner_a = my_pos ^ 1
        partner_b = 3 - my_pos

        barrier_sem = pltpu.get_barrier_semaphore()
        for nbr in (partner_a, partner_b):
            pl.semaphore_signal(
                barrier_sem, inc=1,
                device_id=(nbr,), device_id_type=pl.DeviceIdType.MESH,
            )

        def _exchange(stage, b, partner):
            return pltpu.make_async_remote_copy(
                src_ref=send_ref.at[stage, b],
                dst_ref=recv_ref.at[stage, b],
                send_sem=send_sems.at[stage, b],
                recv_sem=recv_sems.at[stage, b],
                device_id=(partner,),
                device_id_type=pl.DeviceIdType.MESH,
            )

        rdma_a = [_exchange(0, b, partner_a) for b in range(B)]
        rdma_b = [_exchange(1, b, partner_b) for b in range(B)]

        wq = wq_ref[:, :].astype(jnp.bfloat16)
        wo = wo_ref[:, :].astype(jnp.bfloat16)

        for b in range(B):
            xb = x_ref[b, :, :].astype(jnp.bfloat16)
            qb = jnp.dot(xb, wq, preferred_element_type=jnp.float32)
            ctxs = []
            for h in range(HQ_LOCAL):
                qh = qb[:, h * DH:(h + 1) * DH].astype(jnp.bfloat16)
                kh = k_ref[b, :, h, :].astype(jnp.bfloat16)
                vh = v_ref[b, :, h, :].astype(jnp.bfloat16)
                s = jnp.dot(qh, kh.T, preferred_element_type=jnp.float32) * 0.125
                s = s - jnp.max(s, axis=-1, keepdims=True)
                w = jnp.exp(s)
                w = w / jnp.sum(w, axis=-1, keepdims=True)
                ctxs.append(jnp.dot(w.astype(jnp.bfloat16), vh,
                                    preferred_element_type=jnp.float32))
            ctx_b = jnp.concatenate(ctxs, axis=-1)
            pb = jnp.dot(ctx_b.astype(jnp.bfloat16), wo,
                         preferred_element_type=jnp.float32)
            acc_ref[b, :, :] = pb
            send_ref[0, b, :, :] = pb.astype(jnp.bfloat16)
            if b == 0:
                pl.semaphore_wait(barrier_sem, 2)
            rdma_a[b].start()

        for b in range(B):
            rdma_a[b].wait()
            acc = acc_ref[b, :, :] + recv_ref[0, b].astype(jnp.float32)
            acc_ref[b, :, :] = acc
            send_ref[1, b, :, :] = acc.astype(jnp.bfloat16)
            rdma_b[b].start()

        for b in range(B):
            rdma_b[b].wait()
            out_ref[b, :, :] = (acc_ref[b, :, :]
                                + recv_ref[1, b].astype(jnp.float32)
                                ).astype(out_ref.dtype)

    return pl.pallas_call(
        body,
        out_shape=jax.ShapeDtypeStruct((B, SQ, D_MODEL), jnp.float32),
        in_specs=[pl.BlockSpec(memory_space=pltpu.VMEM)] * 5,
        out_specs=pl.BlockSpec(memory_space=pltpu.VMEM),
        scratch_shapes=[
            pltpu.VMEM((B, SQ, D_MODEL), jnp.float32),
            pltpu.VMEM((2, B, SQ, D_MODEL), jnp.bfloat16),
            pltpu.VMEM((2, B, SQ, D_MODEL), jnp.bfloat16),
            pltpu.SemaphoreType.DMA((2, B)),
            pltpu.SemaphoreType.DMA((2, B)),
        ],
        compiler_params=pltpu.CompilerParams(collective_id=0),
    )(x, Wq, K_loc, V_loc, Wo)
